# baseline (device time: 51569 ns/iter reference)
import functools

import jax
import jax.numpy as jnp
from jax import lax
from jax.experimental import pallas as pl
from jax.experimental.pallas import tpu as pltpu

N_DEV = 16
B, H, D = 8, 8, 64
PAGES_PER_DEV = 64
BLOCK = 16
T_LOCAL = PAGES_PER_DEV * BLOCK
LANES = 128


def kernel(Q, K, V, bt, lens):
    my_pos = lax.axis_index("i")
    base = (my_pos * PAGES_PER_DEV).astype(jnp.int32)

    j = jnp.arange(PAGES_PER_DEV, dtype=jnp.int32)
    btm = jnp.where(j[None, :] < lens[:, None], bt, -1)
    w_page = jnp.sum(
        (btm[:, :, None] == (base + j)[None, None, :]).astype(jnp.float32),
        axis=1,
    )
    w_tok = jnp.repeat(w_page, BLOCK, axis=1)

    qp = (Q[:, 0, :, :] * (D ** -0.5)).astype(jnp.float32)
    k3 = K.reshape(T_LOCAL, H, D)
    v3 = V.reshape(T_LOCAL, H, D)

    def body(q_ref, k_ref, v_ref, w_ref, out_ref, comm_ref, send_sems, recv_sems):
        my = lax.axis_index("i")
        left = lax.rem(my + N_DEV - 1, N_DEV)
        right = lax.rem(my + 1, N_DEV)

        barrier = pltpu.get_barrier_semaphore()
        for nbr in (left, right):
            pl.semaphore_signal(
                barrier, inc=1, device_id=(nbr,),
                device_id_type=pl.DeviceIdType.MESH,
            )
        pl.semaphore_wait(barrier, 2)

        w = w_ref[:, :]
        valid = w > 0.0
        neg = jnp.float32(-1e30)

        for h in range(H):
            qh = q_ref[:, h, :]
            kh = k_ref[:, h, :]
            vh = v_ref[:, h, :]
            s = lax.dot_general(
                qh, kh, (((1,), (1,)), ((), ())),
                preferred_element_type=jnp.float32,
            )
            sm = jnp.where(valid, s, neg)
            m = jnp.max(sm, axis=1, keepdims=True)
            p = jnp.exp(sm - m) * w
            l = jnp.sum(p, axis=1, keepdims=True)
            o = lax.dot_general(
                p, vh, (((1,), (0,)), ((), ())),
                preferred_element_type=jnp.float32,
            )
            comm_ref[0, :, h, 0:D] = o
            comm_ref[0, :, h, D:D + 1] = m
            comm_ref[0, :, h, D + 1:D + 2] = l

        o_run = comm_ref[0, :, :, 0:D]
        m_run = comm_ref[0, :, :, D:D + 1]
        l_run = comm_ref[0, :, :, D + 1:D + 2]

        for hop in range(N_DEV - 1):
            rdma = pltpu.make_async_remote_copy(
                src_ref=comm_ref.at[hop],
                dst_ref=comm_ref.at[hop + 1],
                send_sem=send_sems.at[hop],
                recv_sem=recv_sems.at[hop + 1],
                device_id=(right,),
                device_id_type=pl.DeviceIdType.MESH,
            )
            rdma.start()
            rdma.wait()

            o_in = comm_ref[hop + 1, :, :, 0:D]
            m_in = comm_ref[hop + 1, :, :, D:D + 1]
            l_in = comm_ref[hop + 1, :, :, D + 1:D + 2]
            m_new = jnp.maximum(m_run, m_in)
            scale_run = jnp.exp(m_run - m_new)
            scale_in = jnp.exp(m_in - m_new)
            o_run = o_run * scale_run + o_in * scale_in
            l_run = l_run * scale_run + l_in * scale_in
            m_run = m_new

        out_ref[:, 0, :, :] = o_run / l_run

        @functools.partial(pl.run_scoped, exit_bar=pltpu.SemaphoreType.REGULAR)
        def _(exit_bar):
            for nbr in (left, right):
                pl.semaphore_signal(
                    exit_bar, inc=1, device_id=(nbr,),
                    device_id_type=pl.DeviceIdType.MESH,
                )
            pl.semaphore_wait(exit_bar, 2)

    return pl.pallas_call(
        body,
        out_shape=jax.ShapeDtypeStruct((B, 1, H, D), jnp.float32),
        in_specs=[pl.BlockSpec(memory_space=pltpu.VMEM)] * 4,
        out_specs=pl.BlockSpec(memory_space=pltpu.VMEM),
        scratch_shapes=[
            pltpu.VMEM((N_DEV, B, H, LANES), jnp.float32),
            pltpu.SemaphoreType.DMA((N_DEV - 1,)),
            pltpu.SemaphoreType.DMA((N_DEV,)),
        ],
        compiler_params=pltpu.CompilerParams(collective_id=0),
    )(qp, k3, v3, w_tok)


# device time: 25285 ns/iter; 2.0395x vs baseline; 2.0395x over previous
import functools

import jax
import jax.numpy as jnp
from jax import lax
from jax.experimental import pallas as pl
from jax.experimental.pallas import tpu as pltpu

N_DEV = 16
B, H, D = 8, 8, 64
PAGES_PER_DEV = 64
BLOCK = 16
T_LOCAL = PAGES_PER_DEV * BLOCK
LANES = 128


def kernel(Q, K, V, bt, lens):
    my_pos = lax.axis_index("i")
    base = (my_pos * PAGES_PER_DEV).astype(jnp.int32)

    j = jnp.arange(PAGES_PER_DEV, dtype=jnp.int32)
    btm = jnp.where(j[None, :] < lens[:, None], bt, -1)
    w_page = jnp.sum(
        (btm[:, :, None] == (base + j)[None, None, :]).astype(jnp.float32),
        axis=1,
    )
    w_tok = jnp.repeat(w_page, BLOCK, axis=1)

    qp = (Q[:, 0, :, :] * (D ** -0.5)).astype(jnp.float32)
    k3 = K.reshape(T_LOCAL, H, D)
    v3 = V.reshape(T_LOCAL, H, D)

    def body(q_ref, k_ref, v_ref, w_ref, out_ref, comm_ref, send_sems, recv_sems):
        my = lax.axis_index("i")

        barrier = pltpu.get_barrier_semaphore()
        for o in range(1, N_DEV):
            peer = lax.rem(my + o, N_DEV)
            pl.semaphore_signal(
                barrier, inc=1, device_id=(peer,),
                device_id_type=pl.DeviceIdType.MESH,
            )
        pl.semaphore_wait(barrier, N_DEV - 1)

        w = w_ref[:, :]
        valid = w > 0.0
        neg = jnp.float32(-1e30)

        for h in range(H):
            qh = q_ref[:, h, :]
            kh = k_ref[:, h, :]
            vh = v_ref[:, h, :]
            s = lax.dot_general(
                qh, kh, (((1,), (1,)), ((), ())),
                preferred_element_type=jnp.float32,
            )
            sm = jnp.where(valid, s, neg)
            m = jnp.max(sm, axis=1, keepdims=True)
            p = jnp.exp(sm - m) * w
            l = jnp.sum(p, axis=1, keepdims=True)
            o = lax.dot_general(
                p, vh, (((1,), (0,)), ((), ())),
                preferred_element_type=jnp.float32,
            )
            comm_ref[0, :, h, 0:D] = o
            comm_ref[0, :, h, D:D + 1] = m
            comm_ref[0, :, h, D + 1:D + 2] = l

        rdmas = {}
        for o in range(1, N_DEV):
            target = lax.rem(my + o, N_DEV)
            slot = N_DEV - o
            rdmas[o] = pltpu.make_async_remote_copy(
                src_ref=comm_ref.at[0],
                dst_ref=comm_ref.at[slot],
                send_sem=send_sems.at[o],
                recv_sem=recv_sems.at[slot],
                device_id=(target,),
                device_id_type=pl.DeviceIdType.MESH,
            )
            rdmas[o].start()

        o_run = comm_ref[0, :, :, 0:D]
        m_run = comm_ref[0, :, :, D:D + 1]
        l_run = comm_ref[0, :, :, D + 1:D + 2]

        for k in range(1, N_DEV):
            rdmas[N_DEV - k].wait()

            o_in = comm_ref[k, :, :, 0:D]
            m_in = comm_ref[k, :, :, D:D + 1]
            l_in = comm_ref[k, :, :, D + 1:D + 2]
            m_new = jnp.maximum(m_run, m_in)
            scale_run = jnp.exp(m_run - m_new)
            scale_in = jnp.exp(m_in - m_new)
            o_run = o_run * scale_run + o_in * scale_in
            l_run = l_run * scale_run + l_in * scale_in
            m_run = m_new

        out_ref[:, 0, :, :] = o_run / l_run

    return pl.pallas_call(
        body,
        out_shape=jax.ShapeDtypeStruct((B, 1, H, D), jnp.float32),
        in_specs=[pl.BlockSpec(memory_space=pltpu.VMEM)] * 4,
        out_specs=pl.BlockSpec(memory_space=pltpu.VMEM),
        scratch_shapes=[
            pltpu.VMEM((N_DEV, B, H, LANES), jnp.float32),
            pltpu.SemaphoreType.DMA((N_DEV,)),
            pltpu.SemaphoreType.DMA((N_DEV,)),
        ],
        compiler_params=pltpu.CompilerParams(collective_id=0),
    )(qp, k3, v3, w_tok)


# device time: 20680 ns/iter; 2.4937x vs baseline; 1.2227x over previous
import jax
import jax.numpy as jnp
from jax import lax
from jax.experimental import pallas as pl
from jax.experimental.pallas import tpu as pltpu

N_DEV = 16
B, H, D = 8, 8, 64
PAGES_PER_DEV = 64
BLOCK = 16
T_LOCAL = PAGES_PER_DEV * BLOCK
HD = H * D
BH = B * H
LANES = 128


def kernel(Q, K, V, bt, lens):
    my_pos = lax.axis_index("i")
    base = (my_pos * PAGES_PER_DEV).astype(jnp.int32)

    j = jnp.arange(PAGES_PER_DEV, dtype=jnp.int32)
    btm = jnp.where(j[None, :] < lens[:, None], bt, -1)
    w_page = jnp.sum(
        (btm[:, :, None] == (base + j)[None, None, :]).astype(jnp.float32),
        axis=1,
    )
    w_tok = jnp.repeat(w_page, BLOCK, axis=1)
    w_all = jnp.tile(w_tok, (H, 1))

    qp = (Q[:, 0, :, :] * (D ** -0.5)).astype(jnp.float32)
    qbd = (
        jnp.eye(H, dtype=jnp.float32)[:, None, :, None]
        * qp.transpose(1, 0, 2)[:, :, None, :]
    ).reshape(BH, HD)
    kf = K.reshape(T_LOCAL, HD)
    vf = V.reshape(T_LOCAL, HD)

    def body(q_ref, k_ref, v_ref, w_ref, out_ref, comm_ref, send_sems, recv_sems):
        my = lax.axis_index("i")

        barrier = pltpu.get_barrier_semaphore()
        for o in range(1, N_DEV):
            peer = lax.rem(my + o, N_DEV)
            pl.semaphore_signal(
                barrier, inc=1, device_id=(peer,),
                device_id_type=pl.DeviceIdType.MESH,
            )
        pl.semaphore_wait(barrier, N_DEV - 1)

        w = w_ref[:, :]
        s = lax.dot_general(
            q_ref[:, :], k_ref[:, :], (((1,), (1,)), ((), ())),
            preferred_element_type=jnp.float32,
        )
        sm = jnp.where(w > 0.0, s, jnp.float32(-1e30))
        m = jnp.max(sm, axis=1, keepdims=True)
        p = jnp.exp(sm - m) * w
        l = jnp.sum(p, axis=1, keepdims=True)
        obig = lax.dot_general(
            p, v_ref[:, :], (((1,), (0,)), ((), ())),
            preferred_element_type=jnp.float32,
        )
        for h in range(H):
            comm_ref[0, :, h, 0:D] = obig[h * B:(h + 1) * B, h * D:(h + 1) * D]
            comm_ref[0, :, h, D:D + 1] = m[h * B:(h + 1) * B, :]
            comm_ref[0, :, h, D + 1:D + 2] = l[h * B:(h + 1) * B, :]

        rdmas = {}
        for o in range(1, N_DEV):
            target = lax.rem(my + o, N_DEV)
            slot = N_DEV - o
            rdmas[o] = pltpu.make_async_remote_copy(
                src_ref=comm_ref.at[0],
                dst_ref=comm_ref.at[slot],
                send_sem=send_sems.at[o],
                recv_sem=recv_sems.at[slot],
                device_id=(target,),
                device_id_type=pl.DeviceIdType.MESH,
            )
            rdmas[o].start()
        for o in range(1, N_DEV):
            rdmas[o].wait()

        o_all = comm_ref[:, :, :, 0:D]
        m_all = comm_ref[:, :, :, D:D + 1]
        l_all = comm_ref[:, :, :, D + 1:D + 2]
        m_g = jnp.max(m_all, axis=0)
        scale = jnp.exp(m_all - m_g[None])
        o_g = jnp.sum(o_all * scale, axis=0)
        l_g = jnp.sum(l_all * scale, axis=0)

        out_ref[:, 0, :, :] = o_g / l_g

    return pl.pallas_call(
        body,
        out_shape=jax.ShapeDtypeStruct((B, 1, H, D), jnp.float32),
        in_specs=[pl.BlockSpec(memory_space=pltpu.VMEM)] * 4,
        out_specs=pl.BlockSpec(memory_space=pltpu.VMEM),
        scratch_shapes=[
            pltpu.VMEM((N_DEV, B, H, LANES), jnp.float32),
            pltpu.SemaphoreType.DMA((N_DEV,)),
            pltpu.SemaphoreType.DMA((N_DEV,)),
        ],
        compiler_params=pltpu.CompilerParams(collective_id=0),
    )(qbd, kf, vf, w_all)


# device time: 17473 ns/iter; 2.9514x vs baseline; 1.1835x over previous
import jax
import jax.numpy as jnp
from jax import lax
from jax.experimental import pallas as pl
from jax.experimental.pallas import tpu as pltpu

N_DEV = 16
B, H, D = 8, 8, 64
PAGES_PER_DEV = 64
BLOCK = 16
T_LOCAL = PAGES_PER_DEV * BLOCK
HD = H * D
BH = B * H
LANES = 128


def kernel(Q, K, V, bt, lens):
    kf = K.reshape(T_LOCAL, HD)
    vf = V.reshape(T_LOCAL, HD)
    lens2 = lens.reshape(B, 1)

    def body(q_ref, k_ref, v_ref, bt_ref, lens_ref, out_ref,
             comm_ref, send_sems, recv_sems):
        my = lax.axis_index("i")

        barrier = pltpu.get_barrier_semaphore()
        for o in range(1, N_DEV):
            peer = lax.rem(my + o, N_DEV)
            pl.semaphore_signal(
                barrier, inc=1, device_id=(peer,),
                device_id_type=pl.DeviceIdType.MESH,
            )
        pl.semaphore_wait(barrier, N_DEV - 1)

        base_f = lax.convert_element_type(my * PAGES_PER_DEV, jnp.float32)
        jl = lax.broadcasted_iota(jnp.int32, (B, PAGES_PER_DEV), 1)
        btm = jnp.where(jl < lens_ref[:, :], bt_ref[:, :], -1)
        btm_t = btm.astype(jnp.float32).T
        pagef = base_f + jnp.right_shift(
            lax.broadcasted_iota(jnp.int32, (PAGES_PER_DEV, T_LOCAL), 1), 4
        ).astype(jnp.float32)
        rows = []
        for i in range(B):
            cmp = (btm_t[:, i:i + 1] == pagef).astype(jnp.float32)
            rows.append(jnp.sum(cmp, axis=0, keepdims=True))
        w8 = jnp.concatenate(rows, axis=0)
        w = jnp.concatenate([w8] * H, axis=0)

        scale = jnp.float32(D ** -0.5)
        qflat = jnp.concatenate(
            [q_ref[:, 0, h, :] for h in range(H)], axis=0
        ) * scale
        rh = jnp.right_shift(
            lax.broadcasted_iota(jnp.int32, (BH, HD), 0), 3)
        ch = jnp.right_shift(
            lax.broadcasted_iota(jnp.int32, (BH, HD), 1), 6)
        qbd = jnp.concatenate([qflat] * H, axis=1) * (rh == ch).astype(
            jnp.float32)

        s = lax.dot_general(
            qbd, k_ref[:, :], (((1,), (1,)), ((), ())),
            preferred_element_type=jnp.float32,
        )
        sm = jnp.where(w > 0.0, s, jnp.float32(-1e30))
        m = jnp.max(sm, axis=1, keepdims=True)
        p = jnp.exp(sm - m) * w
        l = jnp.sum(p, axis=1, keepdims=True)
        obig = lax.dot_general(
            p, v_ref[:, :], (((1,), (0,)), ((), ())),
            preferred_element_type=jnp.float32,
        )

        for h in range(H):
            comm_ref[0, h * B:(h + 1) * B, 0:D] = obig[
                h * B:(h + 1) * B, h * D:(h + 1) * D].astype(jnp.bfloat16)
        comm_ref[0, :, D:D + 1] = m.astype(jnp.bfloat16)
        comm_ref[0, :, D + 1:D + 2] = l.astype(jnp.bfloat16)

        rdmas = {}
        for o in range(1, N_DEV):
            target = lax.rem(my + o, N_DEV)
            slot = N_DEV - o
            rdmas[o] = pltpu.make_async_remote_copy(
                src_ref=comm_ref.at[0],
                dst_ref=comm_ref.at[slot],
                send_sem=send_sems.at[o],
                recv_sem=recv_sems.at[slot],
                device_id=(target,),
                device_id_type=pl.DeviceIdType.MESH,
            )
            rdmas[o].start()
        for o in range(1, N_DEV):
            rdmas[o].wait()

        o_all = comm_ref[:, :, 0:D].astype(jnp.float32)
        m_all = comm_ref[:, :, D:D + 1].astype(jnp.float32)
        l_all = comm_ref[:, :, D + 1:D + 2].astype(jnp.float32)
        m_g = jnp.max(m_all, axis=0)
        sc = jnp.exp(m_all - m_g[None])
        o_g = jnp.sum(o_all * sc, axis=0)
        l_g = jnp.sum(l_all * sc, axis=0)
        res = o_g / l_g

        for h in range(H):
            out_ref[:, 0, h, :] = res[h * B:(h + 1) * B, :]

    return pl.pallas_call(
        body,
        out_shape=jax.ShapeDtypeStruct((B, 1, H, D), jnp.float32),
        in_specs=[pl.BlockSpec(memory_space=pltpu.VMEM)] * 5,
        out_specs=pl.BlockSpec(memory_space=pltpu.VMEM),
        scratch_shapes=[
            pltpu.VMEM((N_DEV, BH, LANES), jnp.bfloat16),
            pltpu.SemaphoreType.DMA((N_DEV,)),
            pltpu.SemaphoreType.DMA((N_DEV,)),
        ],
        compiler_params=pltpu.CompilerParams(collective_id=0),
    )(Q, kf, vf, bt, lens2)
